# baseline (speedup 1.0000x reference)
import functools

import jax
import jax.numpy as jnp
from jax import lax
from jax.experimental import pallas as pl
from jax.experimental.pallas import tpu as pltpu
from jax.experimental.pallas import tpu_sc as plsc

_G = 128
_NC = 2
_NS = 16
_NW = _NC * _NS
_CHUNK = 128
_GC = 16


def _leaky(v):
    return jnp.where(v >= 0, v, 0.01 * v)


def _mm2_body(x_ref, wn_ref, ws_ref, xw_ref, xs_ref):
    x = x_ref[...]
    xw_ref[...] = jnp.dot(x, wn_ref[...], preferred_element_type=jnp.float32)
    xs_ref[...] = jnp.dot(x, ws_ref[...], preferred_element_type=jnp.float32)


def _mid_body(xs_ref, p_ref, b_ref, g_ref, be_ref, wn_ref, ws_ref,
              xw_ref, xs1_ref):
    h = xs_ref[...] + p_ref[0] + p_ref[1] + b_ref[...]
    mu = jnp.mean(h, axis=0, keepdims=True)
    d = h - mu
    var = jnp.mean(d * d, axis=0, keepdims=True)
    hn = g_ref[...] * d * lax.rsqrt(var + 1e-5) + be_ref[...]
    h = _leaky(hn)
    xw_ref[...] = jnp.dot(h, wn_ref[...], preferred_element_type=jnp.float32)
    xs1_ref[...] = jnp.dot(h, ws_ref[...], preferred_element_type=jnp.float32)


def _final_body(xs_ref, p_ref, b_ref, g_ref, be_ref, bi_ref,
                wr0_ref, br0_ref, wr1_ref, br1_ref, wo_ref, bo_ref, out_ref):
    h = xs_ref[...] + p_ref[0] + p_ref[1] + b_ref[...]
    mu = jnp.mean(h, axis=0, keepdims=True)
    d = h - mu
    var = jnp.mean(d * d, axis=0, keepdims=True)
    hn = g_ref[...] * d * lax.rsqrt(var + 1e-5) + be_ref[...]
    h = _leaky(hn)
    gids = lax.broadcasted_iota(jnp.int32, (_G, 1), 0)
    mask = (bi_ref[...] == gids).astype(jnp.float32)
    sums = jnp.dot(mask, h, preferred_element_type=jnp.float32)
    cnts = jnp.sum(mask, axis=1, keepdims=True)
    pooled = sums / jnp.maximum(cnts, 1.0)
    r = _leaky(jnp.dot(pooled, wr0_ref[...],
                       preferred_element_type=jnp.float32) + br0_ref[...])
    r = _leaky(jnp.dot(r, wr1_ref[...],
                       preferred_element_type=jnp.float32) + br1_ref[...])
    out_ref[...] = jnp.dot(r, wo_ref[...],
                           preferred_element_type=jnp.float32) + bo_ref[...]


@functools.lru_cache(maxsize=None)
def _build_sc_agg(n_nodes, nchunk, feat):
    npad = -(-n_nodes // 128) * 128
    if npad == n_nodes:
        npad += 128
    zrows = npad // _NS
    orows = zrows
    mesh = plsc.VectorSubcoreMesh(core_axis_name="c", subcore_axis_name="s")

    nchunk, nch0, nch1 = nchunk

    @functools.partial(
        pl.kernel,
        mesh=mesh,
        out_type=jax.ShapeDtypeStruct((_NC, npad, feat), jnp.float32),
        scratch_types=[
            pltpu.VMEM((nchunk, _CHUNK), jnp.int32),
            pltpu.VMEM((nchunk, _CHUNK), jnp.int32),
            pltpu.VMEM((_CHUNK, feat), jnp.float32),
            pltpu.VMEM_SHARED((npad, feat), jnp.float32),
            pltpu.SemaphoreType.DMA,
        ],
    )
    def agg(xw_hbm, src_hbm, dst_hbm, zeros_hbm, out_hbm,
            src_v, dst_v, rows_v, acc_sh, sem):
        c = lax.axis_index("c")
        s = lax.axis_index("s")
        wid = c * _NS + s
        pltpu.sync_copy(zeros_hbm.at[pl.ds(s * zrows, zrows)],
                        acc_sh.at[pl.ds(s * zrows, zrows)])
        pltpu.sync_copy(src_hbm.at[wid], src_v)
        pltpu.sync_copy(dst_hbm.at[wid], dst_v)
        plsc.subcore_barrier()

        def body(j, carry):
            pltpu.async_copy(xw_hbm.at[src_v.at[j]], rows_v, sem).wait()
            pltpu.sync_copy(rows_v, acc_sh.at[dst_v.at[j]], add=True)
            return carry

        nch = jnp.where(c == 0, nch0, nch1)
        lax.fori_loop(0, nch, body, 0)
        plsc.subcore_barrier()
        pltpu.sync_copy(acc_sh.at[pl.ds(s * orows, orows)],
                        out_hbm.at[c, pl.ds(s * orows, orows)])

    return agg


def _sc_aggregate(xw, src3, dst3, zeros, h_dim, nch0, nch1):
    n = xw.shape[0]
    p = _build_sc_agg(n, (src3.shape[1], nch0, nch1), xw.shape[1])(
        xw, src3, dst3, zeros)
    return p[:, :n, :h_dim]


def kernel(x, edge_index, batch_index,
           W_self0, W_nei0, b0, gamma0, beta0,
           W_self1, W_nei1, b1, gamma1, beta1,
           W_r0, b_r0, W_r1, b_r1, W_out, b_out):
    n, _ = x.shape
    h_dim = W_nei0.shape[1]
    e = edge_index.shape[1]

    src = edge_index[0].astype(jnp.int32)
    dst = edge_index[1].astype(jnp.int32)
    per_pair = -(-e // (_NS * _CHUNK))
    nch0 = max(1, round(per_pair * 0.66))
    nch1 = per_pair - nch0
    nch_max = max(nch0, nch1)
    epad = _NS * per_pair * _CHUNK
    if epad > e:
        src = jnp.concatenate([src, jnp.zeros((epad - e,), jnp.int32)])
        dst = jnp.concatenate([dst, jnp.full((epad - e,), n, jnp.int32)])
    e0 = _NS * nch0 * _CHUNK

    def slab(a, cnt, off, fill):
        m = a[off:off + _NS * cnt * _CHUNK].reshape(_NS, cnt, _CHUNK)
        if cnt < nch_max:
            m = jnp.pad(m, ((0, 0), (0, nch_max - cnt), (0, 0)),
                        constant_values=fill)
        return m

    src3 = jnp.concatenate([slab(src, nch0, 0, 0), slab(src, nch1, e0, 0)])
    dst3 = jnp.concatenate([slab(dst, nch0, 0, n), slab(dst, nch1, e0, n)])
    acc_rows = -(-n // 128) * 128
    if acc_rows == n:
        acc_rows += 128
    fpad = 128
    zeros = jnp.zeros((acc_rows, fpad), jnp.float32)
    wn0p = jnp.pad(W_nei0, ((0, 0), (0, fpad - h_dim)))
    wn1p = jnp.pad(W_nei1, ((0, 0), (0, fpad - h_dim)))

    b0r = b0.reshape(1, -1)
    b1r = b1.reshape(1, -1)

    xw0, xs0 = pl.pallas_call(
        _mm2_body,
        out_shape=[jax.ShapeDtypeStruct((n, fpad), jnp.float32),
                   jax.ShapeDtypeStruct((n, h_dim), jnp.float32)],
    )(x, wn0p, W_self0)

    p0 = _sc_aggregate(xw0, src3, dst3, zeros, h_dim, nch0, nch1)

    xw1, xs1 = pl.pallas_call(
        _mid_body,
        out_shape=[jax.ShapeDtypeStruct((n, fpad), jnp.float32),
                   jax.ShapeDtypeStruct((n, h_dim), jnp.float32)],
    )(xs0, p0, b0r, gamma0.reshape(1, -1),
      beta0.reshape(1, -1), wn1p, W_self1)

    p1 = _sc_aggregate(xw1, src3, dst3, zeros, h_dim, nch0, nch1)

    preds = pl.pallas_call(
        _final_body,
        out_shape=jax.ShapeDtypeStruct((_G, 1), jnp.float32),
    )(xs1, p1, b1r, gamma1.reshape(1, -1), beta1.reshape(1, -1),
      batch_index.astype(jnp.int32).reshape(1, n),
      W_r0, b_r0.reshape(1, -1), W_r1, b_r1.reshape(1, -1),
      W_out, b_out.reshape(1, -1))
    return preds.astype(jnp.float32)


# speedup vs baseline: 1.0370x; 1.0370x over previous
import functools

import jax
import jax.numpy as jnp
from jax import lax
from jax.experimental import pallas as pl
from jax.experimental.pallas import tpu as pltpu
from jax.experimental.pallas import tpu_sc as plsc

_G = 128
_NC = 2
_NS = 16
_NW = _NC * _NS
_CHUNK = 128
_GC = 16


def _leaky(v):
    return jnp.where(v >= 0, v, 0.01 * v)


def _mm2_body(x_ref, wn_ref, ws_ref, xw_ref, xs_ref):
    x = x_ref[...]
    xw_ref[...] = jnp.dot(x, wn_ref[...], preferred_element_type=jnp.float32)
    xs_ref[...] = jnp.dot(x, ws_ref[...], preferred_element_type=jnp.float32)


def _mid_body(xs_ref, p_ref, b_ref, g_ref, be_ref, wn_ref, ws_ref,
              xw_ref, xs1_ref):
    h = xs_ref[...] + p_ref[0] + p_ref[1] + b_ref[...]
    mu = jnp.mean(h, axis=0, keepdims=True)
    d = h - mu
    var = jnp.mean(d * d, axis=0, keepdims=True)
    hn = g_ref[...] * d * lax.rsqrt(var + 1e-5) + be_ref[...]
    h = _leaky(hn)
    xw_ref[...] = jnp.dot(h, wn_ref[...], preferred_element_type=jnp.float32)
    xs1_ref[...] = jnp.dot(h, ws_ref[...], preferred_element_type=jnp.float32)


def _final_body(xs_ref, p_ref, b_ref, g_ref, be_ref, bi_ref,
                wr0_ref, br0_ref, wr1_ref, br1_ref, wo_ref, bo_ref, out_ref):
    h = xs_ref[...] + p_ref[0] + p_ref[1] + b_ref[...]
    mu = jnp.mean(h, axis=0, keepdims=True)
    d = h - mu
    var = jnp.mean(d * d, axis=0, keepdims=True)
    hn = g_ref[...] * d * lax.rsqrt(var + 1e-5) + be_ref[...]
    h = _leaky(hn)
    gids = lax.broadcasted_iota(jnp.int32, (_G, 1), 0)
    mask = (bi_ref[...] == gids).astype(jnp.float32)
    sums = jnp.dot(mask, h, preferred_element_type=jnp.float32)
    cnts = jnp.sum(mask, axis=1, keepdims=True)
    pooled = sums / jnp.maximum(cnts, 1.0)
    r = _leaky(jnp.dot(pooled, wr0_ref[...],
                       preferred_element_type=jnp.float32) + br0_ref[...])
    r = _leaky(jnp.dot(r, wr1_ref[...],
                       preferred_element_type=jnp.float32) + br1_ref[...])
    out_ref[...] = jnp.dot(r, wo_ref[...],
                           preferred_element_type=jnp.float32) + bo_ref[...]


@functools.lru_cache(maxsize=None)
def _build_sc_agg(n_nodes, nchunk, feat):
    npad = -(-n_nodes // 128) * 128
    if npad == n_nodes:
        npad += 128
    zrows = npad // _NS
    orows = zrows
    mesh = plsc.VectorSubcoreMesh(core_axis_name="c", subcore_axis_name="s")

    nchunk, nch0, nch1 = nchunk

    @functools.partial(
        pl.kernel,
        mesh=mesh,
        out_type=jax.ShapeDtypeStruct((_NC, npad, feat), jnp.float32),
        scratch_types=[
            pltpu.VMEM((nchunk, _CHUNK), jnp.int32),
            pltpu.VMEM((nchunk, _CHUNK), jnp.int32),
            pltpu.VMEM((_CHUNK, feat), jnp.float32),
            pltpu.VMEM_SHARED((npad, feat), jnp.float32),
            pltpu.SemaphoreType.DMA,
        ],
    )
    def agg(xw_hbm, src_hbm, dst_hbm, zeros_hbm, out_hbm,
            src_v, dst_v, rows_v, acc_sh, sem):
        c = lax.axis_index("c")
        s = lax.axis_index("s")
        wid = c * _NS + s
        pltpu.sync_copy(zeros_hbm.at[pl.ds(s * zrows, zrows)],
                        acc_sh.at[pl.ds(s * zrows, zrows)])
        pltpu.sync_copy(src_hbm.at[wid], src_v)
        pltpu.sync_copy(dst_hbm.at[wid], dst_v)
        plsc.subcore_barrier()

        def body(j, carry):
            pltpu.async_copy(xw_hbm.at[src_v.at[j]], rows_v, sem).wait()
            pltpu.sync_copy(rows_v, acc_sh.at[dst_v.at[j]], add=True)
            return carry

        nch = jnp.where(c == 0, nch0, nch1)
        lax.fori_loop(0, nch, body, 0)
        plsc.subcore_barrier()
        pltpu.sync_copy(acc_sh.at[pl.ds(s * orows, orows)],
                        out_hbm.at[c, pl.ds(s * orows, orows)])

    return agg


def _sc_aggregate(xw, src3, dst3, zeros, h_dim, nch0, nch1):
    n = xw.shape[0]
    p = _build_sc_agg(n, (src3.shape[1], nch0, nch1), xw.shape[1])(
        xw, src3, dst3, zeros)
    return p[:, :n, :h_dim]


def kernel(x, edge_index, batch_index,
           W_self0, W_nei0, b0, gamma0, beta0,
           W_self1, W_nei1, b1, gamma1, beta1,
           W_r0, b_r0, W_r1, b_r1, W_out, b_out):
    n, _ = x.shape
    h_dim = W_nei0.shape[1]
    e = edge_index.shape[1]

    src = edge_index[0].astype(jnp.int32)
    dst = edge_index[1].astype(jnp.int32)
    per_pair = -(-e // (_NS * _CHUNK))
    nch0 = max(1, round(per_pair * 0.63))
    nch1 = per_pair - nch0
    nch_max = max(nch0, nch1)
    epad = _NS * per_pair * _CHUNK
    if epad > e:
        src = jnp.concatenate([src, jnp.zeros((epad - e,), jnp.int32)])
        dst = jnp.concatenate([dst, jnp.full((epad - e,), n, jnp.int32)])
    e0 = _NS * nch0 * _CHUNK

    def slab(a, cnt, off, fill):
        m = a[off:off + _NS * cnt * _CHUNK].reshape(_NS, cnt, _CHUNK)
        if cnt < nch_max:
            m = jnp.pad(m, ((0, 0), (0, nch_max - cnt), (0, 0)),
                        constant_values=fill)
        return m

    src3 = jnp.concatenate([slab(src, nch0, 0, 0), slab(src, nch1, e0, 0)])
    dst3 = jnp.concatenate([slab(dst, nch0, 0, n), slab(dst, nch1, e0, n)])
    acc_rows = -(-n // 128) * 128
    if acc_rows == n:
        acc_rows += 128
    fpad = 128
    zeros = jnp.zeros((acc_rows, fpad), jnp.float32)
    wn0p = jnp.pad(W_nei0, ((0, 0), (0, fpad - h_dim)))
    wn1p = jnp.pad(W_nei1, ((0, 0), (0, fpad - h_dim)))

    b0r = b0.reshape(1, -1)
    b1r = b1.reshape(1, -1)

    xw0, xs0 = pl.pallas_call(
        _mm2_body,
        out_shape=[jax.ShapeDtypeStruct((n, fpad), jnp.float32),
                   jax.ShapeDtypeStruct((n, h_dim), jnp.float32)],
    )(x, wn0p, W_self0)

    p0 = _sc_aggregate(xw0, src3, dst3, zeros, h_dim, nch0, nch1)

    xw1, xs1 = pl.pallas_call(
        _mid_body,
        out_shape=[jax.ShapeDtypeStruct((n, fpad), jnp.float32),
                   jax.ShapeDtypeStruct((n, h_dim), jnp.float32)],
    )(xs0, p0, b0r, gamma0.reshape(1, -1),
      beta0.reshape(1, -1), wn1p, W_self1)

    p1 = _sc_aggregate(xw1, src3, dst3, zeros, h_dim, nch0, nch1)

    preds = pl.pallas_call(
        _final_body,
        out_shape=jax.ShapeDtypeStruct((_G, 1), jnp.float32),
    )(xs1, p1, b1r, gamma1.reshape(1, -1), beta1.reshape(1, -1),
      batch_index.astype(jnp.int32).reshape(1, n),
      W_r0, b_r0.reshape(1, -1), W_r1, b_r1.reshape(1, -1),
      W_out, b_out.reshape(1, -1))
    return preds.astype(jnp.float32)
